# baseline (speedup 1.0000x reference)
import functools

import jax
import jax.numpy as jnp
from jax import lax
from jax.experimental import pallas as pl
from jax.experimental.pallas import tpu as pltpu
from jax.experimental.pallas import tpu_sc as plsc

N, C, G, E, H = 10000, 32, 128, 320000, 8
NPAD = 10240
EPB = 128
ROWS = 2560
EPAD = ROWS * EPB
NTILES = 32
TPB = ROWS // NTILES

_mesh = plsc.VectorSubcoreMesh(core_axis_name="c", subcore_axis_name="s")
_dn = (((1,), (1,)), ((), ()))


def _tc_proj_body(x_ref, wl_ref, bl_ref, wr_ref, br_ref, xl_ref, xr_ref):
    x = x_ref[...]
    xl_ref[...] = lax.dot_general(x, wl_ref[...], _dn,
                                  preferred_element_type=jnp.float32) + bl_ref[...]
    xr_ref[...] = lax.dot_general(x, wr_ref[...], _dn,
                                  preferred_element_type=jnp.float32) + br_ref[...]


_tc_proj = pl.pallas_call(
    _tc_proj_body,
    out_shape=[jax.ShapeDtypeStruct((N, H), jnp.float32),
               jax.ShapeDtypeStruct((N, H), jnp.float32)],
)


def _tc_dense_body(w_ref, x_ref, mu_ref, var_ref, s_ref, p_ref, q_ref, ll_ref):
    w = w_ref[...]
    m = jnp.max(w, axis=1, keepdims=True)
    e = jnp.exp(w - m)
    p = e / jnp.sum(e, axis=1, keepdims=True)
    p_ref[...] = p
    q_ref[...] = jnp.log(p + 1e-8)
    iv = 1.0 / var_ref[...]
    x = x_ref[...]
    xv = x * iv
    A = lax.dot_general(xv, mu_ref[...], _dn,
                        preferred_element_type=jnp.float32)
    a = jnp.sum(x * xv, axis=1, keepdims=True)
    mu2 = mu_ref[...] * mu_ref[...]
    qrow = lax.dot_general(iv, mu2, _dn,
                           preferred_element_type=jnp.float32)
    s = s_ref[...]
    F = -0.5 * (a - 2.0 * s * A + (s * s) * qrow)
    ll_ref[...] = (jnp.sum(p * F) * (1.0 / N)).reshape(1, 1)


_tc_dense = pl.pallas_call(
    _tc_dense_body,
    out_shape=[jax.ShapeDtypeStruct((N, C), jnp.float32),
               jax.ShapeDtypeStruct((N, C), jnp.float32),
               jax.ShapeDtypeStruct((1, 1), jnp.float32)],
)


@functools.partial(
    pl.kernel,
    out_type=[jax.ShapeDtypeStruct((ROWS, EPB), jnp.float32),
              jax.ShapeDtypeStruct((2, NPAD), jnp.float32)],
    mesh=_mesh,
    compiler_params=pltpu.CompilerParams(needs_layout_passes=False, use_tc_tiling_on_sc=False),
    scratch_types=[
        pltpu.VMEM((TPB, EPB), jnp.int32),
        pltpu.VMEM((TPB, EPB), jnp.int32),
        pltpu.VMEM((TPB, EPB), jnp.float32),
        [pltpu.VMEM((EPB, H), jnp.float32) for _ in range(8)],
        [pltpu.VMEM((EPB, H), jnp.float32) for _ in range(8)],
        pltpu.VMEM((16,), jnp.float32),
        pltpu.VMEM((NPAD // 16,), jnp.float32),
        pltpu.VMEM_SHARED((NPAD,), jnp.float32),
        [pltpu.SemaphoreType.DMA for _ in range(8)],
        pltpu.SemaphoreType.DMA,
    ],
)
def _sc_edge1(src_hbm, dst_hbm, xl_hbm, xr_hbm, att_hbm, ex_hbm, spart_hbm,
              src_v, dst_v, ex_v, xls, xrs, att_v, zbuf, s_sh, sems, sem_sc):
    cid = lax.axis_index("c")
    sid = lax.axis_index("s")
    wid = cid * 16 + sid
    base = wid * TPB
    nsub = NPAD // 16

    pltpu.sync_copy(src_hbm.at[pl.ds(base, TPB)], src_v)
    pltpu.sync_copy(dst_hbm.at[pl.ds(base, TPB)], dst_v)
    pltpu.sync_copy(att_hbm, att_v)

    iota = lax.iota(jnp.int32, 16)
    zero16 = jnp.zeros((16,), jnp.float32)

    def _zero(i, carry):
        zbuf[pl.ds(i * 16, 16)] = zero16
        return carry

    lax.fori_loop(0, nsub // 16, _zero, 0)
    pltpu.sync_copy(zbuf, s_sh.at[pl.ds(sid * nsub, nsub)])
    plsc.subcore_barrier()

    def _fire(b, xs, rs, sem):
        pltpu.async_copy(xl_hbm.at[src_v.at[b]], xs, sem)
        pltpu.async_copy(xr_hbm.at[dst_v.at[b]], rs, sem)

    def _drain(xs, rs, sem):
        pltpu.make_async_copy(xl_hbm.at[pl.ds(0, EPB)], xs, sem).wait()
        pltpu.make_async_copy(xl_hbm.at[pl.ds(0, EPB)], rs, sem).wait()

    def _compute(b, xs, rs):
        grow = base + b
        att_full = att_v[...]

        def _group(k, carry):
            rows = iota + k * 16
            acc_a = jnp.zeros((16,), jnp.float32)
            acc_b = jnp.zeros((16,), jnp.float32)
            for h in range(H):
                hsp = jnp.full((16,), h, jnp.int32)
                av = att_full[h]
                v = plsc.load_gather(xs, [rows, hsp]) + plsc.load_gather(rs, [rows, hsp])
                acc_a = acc_a + av * v
                acc_b = acc_b + av * jnp.abs(v)
            exv = jnp.exp(0.6 * acc_a + 0.4 * acc_b)
            ids = iota + (grow * EPB + k * 16)
            exv = jnp.where(ids < E, exv, 0.0)
            ex_v[b, pl.ds(k * 16, 16)] = exv
            return carry

        lax.fori_loop(0, EPB // 16, _group, 0)
        pltpu.async_copy(ex_v.at[b], s_sh.at[dst_v.at[b]], sem_sc, add=True)

    for j in range(7):
        _fire(j, xls[j], xrs[j], sems[j])

    def _loop(g, carry):
        for j in range(8):
            b = 8 * g + j
            jf = (j + 7) % 8

            @pl.when(b + 7 < TPB)
            def _():
                _fire(b + 7, xls[jf], xrs[jf], sems[jf])

            _drain(xls[j], xrs[j], sems[j])
            _compute(b, xls[j], xrs[j])
        return carry

    lax.fori_loop(0, TPB // 8, _loop, 0)

    def _drain_sc(b, carry):
        pltpu.make_async_copy(ex_v.at[0], s_sh.at[dst_v.at[0]], sem_sc).wait()
        return carry

    lax.fori_loop(0, TPB, _drain_sc, 0)
    pltpu.sync_copy(ex_v, ex_hbm.at[pl.ds(base, TPB)])
    plsc.subcore_barrier()
    pltpu.sync_copy(s_sh.at[pl.ds(sid * nsub, nsub)],
                    spart_hbm.at[cid, pl.ds(sid * nsub, nsub)])


@functools.partial(
    pl.kernel,
    out_type=jax.ShapeDtypeStruct((NTILES, 16), jnp.float32),
    mesh=_mesh,
    compiler_params=pltpu.CompilerParams(needs_layout_passes=False, use_tc_tiling_on_sc=False),
    scratch_types=[
        pltpu.VMEM((TPB, EPB), jnp.int32),
        pltpu.VMEM((TPB, EPB), jnp.int32),
        pltpu.VMEM((TPB, EPB), jnp.float32),
        pltpu.VMEM((NPAD,), jnp.float32),
        pltpu.VMEM((NPAD,), jnp.float32),
        [pltpu.VMEM((EPB, C), jnp.float32) for _ in range(8)],
        [pltpu.VMEM((EPB, C), jnp.float32) for _ in range(8)],
        pltpu.VMEM((16,), jnp.float32),
        [pltpu.SemaphoreType.DMA for _ in range(8)],
    ],
)
def _sc_edge2(src_hbm, dst_hbm, ex_hbm, spart_hbm, p_hbm, q_hbm, out_hbm,
              src_v, dst_v, ex_v, s_v, st_v, pss, qss, orow, sems):
    cid = lax.axis_index("c")
    sid = lax.axis_index("s")
    wid = cid * 16 + sid
    base = wid * TPB

    pltpu.sync_copy(src_hbm.at[pl.ds(base, TPB)], src_v)
    pltpu.sync_copy(dst_hbm.at[pl.ds(base, TPB)], dst_v)
    pltpu.sync_copy(ex_hbm.at[pl.ds(base, TPB)], ex_v)
    pltpu.sync_copy(spart_hbm.at[0], s_v)
    pltpu.sync_copy(spart_hbm.at[1], st_v)

    iota = lax.iota(jnp.int32, 16)

    def _sum(i, carry):
        sl = pl.ds(i * 16, 16)
        s_v[sl] = s_v[sl] + st_v[sl] + 1e-16
        return carry

    lax.fori_loop(0, NPAD // 16, _sum, 0)

    def _fire(b, ps, qs, sem):
        pltpu.async_copy(p_hbm.at[src_v.at[b]], ps, sem)
        pltpu.async_copy(q_hbm.at[dst_v.at[b]], qs, sem)

    def _drain(ps, qs, sem):
        pltpu.make_async_copy(p_hbm.at[pl.ds(0, EPB)], ps, sem).wait()
        pltpu.make_async_copy(p_hbm.at[pl.ds(0, EPB)], qs, sem).wait()

    def _compute(b, ps, qs, acc):
        def _group(k, acc):
            sl = pl.ds(k * 16, 16)
            rows = iota + k * 16
            sv = plsc.load_gather(s_v, [dst_v[b, sl]])
            alpha = ex_v[b, sl] / sv
            d = [jnp.zeros((16,), jnp.float32) for _ in range(4)]
            for c in range(C):
                csp = jnp.full((16,), c, jnp.int32)
                pc = plsc.load_gather(ps, [rows, csp])
                qc = plsc.load_gather(qs, [rows, csp])
                d[c % 4] = d[c % 4] + pc * qc
            return acc + alpha * ((d[0] + d[1]) + (d[2] + d[3]))

        return lax.fori_loop(0, EPB // 16, _group, acc)

    for j in range(7):
        _fire(j, pss[j], qss[j], sems[j])

    def _loop(g, acc):
        for j in range(8):
            b = 8 * g + j
            jf = (j + 7) % 8

            @pl.when(b + 7 < TPB)
            def _():
                _fire(b + 7, pss[jf], qss[jf], sems[jf])

            _drain(pss[j], qss[j], sems[j])
            acc = _compute(b, pss[j], qss[j], acc)
        return acc

    acc = lax.fori_loop(0, TPB // 8, _loop, jnp.zeros((16,), jnp.float32))
    orow[...] = acc * (-1.0 / N)
    pltpu.sync_copy(orow, out_hbm.at[wid])


def kernel(X, Mu, Var, edge_index, W, S, lin_l_w, lin_l_b, lin_r_w, lin_r_b, att):
    xl, xr = _tc_proj(X, lin_l_w, lin_l_b.reshape(1, H), lin_r_w,
                      lin_r_b.reshape(1, H))
    P, Q, ll = _tc_dense(W, X, Mu, Var.reshape(1, G), S)
    pad = EPAD - E
    src2 = jnp.concatenate(
        [edge_index[0], jnp.zeros((pad,), jnp.int32)]).reshape(ROWS, EPB)
    dst2 = jnp.concatenate(
        [edge_index[1], jnp.zeros((pad,), jnp.int32)]).reshape(ROWS, EPB)
    att16 = jnp.pad(att, (0, 16 - H))
    ex, spart = _sc_edge1(src2, dst2, xl, xr, att16)
    ce_part = _sc_edge2(src2, dst2, ex, spart, P, Q)
    return (ll[0, 0], jnp.sum(ce_part), P)


# speedup vs baseline: 2.5455x; 2.5455x over previous
import functools

import jax
import jax.numpy as jnp
from jax import lax
from jax.experimental import pallas as pl
from jax.experimental.pallas import tpu as pltpu
from jax.experimental.pallas import tpu_sc as plsc

N, C, G, E, H = 10000, 32, 128, 320000, 8
NPAD = 10240
EPB = 128
ROWS = 2560
EPAD = ROWS * EPB
NTILES = 32
TPB = ROWS // NTILES
CH = 1024
EQ_EDGES = EPAD // 4
NCH = EQ_EDGES // CH

_mesh = plsc.VectorSubcoreMesh(core_axis_name="c", subcore_axis_name="s")
_params = pltpu.CompilerParams(needs_layout_passes=False,
                               use_tc_tiling_on_sc=False)
_dn = (((1,), (1,)), ((), ()))


def _tc_proj_body(x_ref, wl_ref, bl_ref, wr_ref, br_ref, xl_ref, xr_ref):
    x = x_ref[...]
    xl_ref[...] = lax.dot_general(x, wl_ref[...], _dn,
                                  preferred_element_type=jnp.float32) + bl_ref[...]
    xr_ref[...] = lax.dot_general(x, wr_ref[...], _dn,
                                  preferred_element_type=jnp.float32) + br_ref[...]


_tc_proj = pl.pallas_call(
    _tc_proj_body,
    out_shape=[jax.ShapeDtypeStruct((N, H), jnp.float32),
               jax.ShapeDtypeStruct((N, H), jnp.float32)],
)


def _tc_dense_body(w_ref, wt_ref, x_ref, mu_ref, var_ref, s_ref,
                   p_ref, pt_ref, qt_ref, ll_ref):
    w = w_ref[...]
    m = jnp.max(w, axis=1, keepdims=True)
    e = jnp.exp(w - m)
    p = e / jnp.sum(e, axis=1, keepdims=True)
    p_ref[...] = p
    wt = wt_ref[...]
    mt = jnp.max(wt, axis=0, keepdims=True)
    et = jnp.exp(wt - mt)
    pt = et / jnp.sum(et, axis=0, keepdims=True)
    pt_ref[...] = pt
    qt_ref[...] = jnp.log(pt + 1e-8)
    iv = 1.0 / var_ref[...]
    x = x_ref[...]
    xv = x * iv
    A = lax.dot_general(xv, mu_ref[...], _dn,
                        preferred_element_type=jnp.float32)
    a = jnp.sum(x * xv, axis=1, keepdims=True)
    mu2 = mu_ref[...] * mu_ref[...]
    qrow = lax.dot_general(iv, mu2, _dn,
                           preferred_element_type=jnp.float32)
    s = s_ref[...]
    F = -0.5 * (a - 2.0 * s * A + (s * s) * qrow)
    ll_ref[...] = (jnp.sum(p * F) * (1.0 / N)).reshape(1, 1)


_tc_dense = pl.pallas_call(
    _tc_dense_body,
    out_shape=[jax.ShapeDtypeStruct((N, C), jnp.float32),
               jax.ShapeDtypeStruct((C, N), jnp.float32),
               jax.ShapeDtypeStruct((C, N), jnp.float32),
               jax.ShapeDtypeStruct((1, 1), jnp.float32)],
)


@functools.partial(
    pl.kernel,
    out_type=[jax.ShapeDtypeStruct((ROWS, EPB), jnp.float32),
              jax.ShapeDtypeStruct((2, NPAD), jnp.float32)],
    mesh=_mesh,
    compiler_params=_params,
    scratch_types=[
        pltpu.VMEM((N, H), jnp.float32),
        pltpu.VMEM((TPB, EPB), jnp.int32),
        pltpu.VMEM((TPB, EPB), jnp.int32),
        pltpu.VMEM((TPB, EPB), jnp.float32),
        pltpu.VMEM((16,), jnp.float32),
        pltpu.VMEM((NPAD // 16,), jnp.float32),
        pltpu.VMEM_SHARED((NPAD,), jnp.float32),
        pltpu.SemaphoreType.DMA,
    ],
)
def _sc_edge1(src_hbm, dst_hbm, xlr_hbm, att_hbm, ex_hbm, spart_hbm,
              xlr_v, src_v, dst_v, ex_v, att_v, zbuf, s_sh, sem_sc):
    cid = lax.axis_index("c")
    sid = lax.axis_index("s")
    wid = cid * 16 + sid
    base = wid * TPB
    nsub = NPAD // 16

    pltpu.sync_copy(xlr_hbm, xlr_v)
    pltpu.sync_copy(src_hbm.at[pl.ds(base, TPB)], src_v)
    pltpu.sync_copy(dst_hbm.at[pl.ds(base, TPB)], dst_v)
    pltpu.sync_copy(att_hbm, att_v)

    iota = lax.iota(jnp.int32, 16)
    zero16 = jnp.zeros((16,), jnp.float32)

    def _zero(i, carry):
        zbuf[pl.ds(i * 16, 16)] = zero16
        return carry

    lax.fori_loop(0, nsub // 16, _zero, 0)
    pltpu.sync_copy(zbuf, s_sh.at[pl.ds(sid * nsub, nsub)])
    plsc.subcore_barrier()

    att_full = att_v[...]
    bf = jnp.bfloat16
    fmt = plsc.PackFormat.INTERLEAVED

    def _row(b, carry):
        grow = base + b

        def _group(k, c2):
            sl = pl.ds(k * 16, 16)
            srcv = src_v[b, sl]
            dstv = dst_v[b, sl]
            acc_a = jnp.zeros((16,), jnp.float32)
            acc_b = jnp.zeros((16,), jnp.float32)
            for w in range(4):
                wl = plsc.load_gather(xlr_v, [srcv, jnp.full((16,), w, jnp.int32)])
                wr = plsc.load_gather(xlr_v, [dstv, jnp.full((16,), w + 4, jnp.int32)])
                la, lb = plsc.unpack(plsc.bitcast(wl, bf), format=fmt)
                ra, rb = plsc.unpack(plsc.bitcast(wr, bf), format=fmt)
                v0 = la + ra
                v1 = lb + rb
                a0 = att_full[2 * w]
                a1 = att_full[2 * w + 1]
                acc_a = acc_a + a0 * v0 + a1 * v1
                acc_b = acc_b + a0 * jnp.abs(v0) + a1 * jnp.abs(v1)
            exv = jnp.exp(0.6 * acc_a + 0.4 * acc_b)
            ids = iota + (grow * EPB + k * 16)
            exv = jnp.where(ids < E, exv, 0.0)
            ex_v[b, sl] = exv
            return c2

        lax.fori_loop(0, EPB // 16, _group, 0)
        pltpu.async_copy(ex_v.at[b], s_sh.at[dst_v.at[b]], sem_sc, add=True)
        return carry

    lax.fori_loop(0, TPB, _row, 0)

    def _drain_sc(b, carry):
        pltpu.make_async_copy(ex_v.at[0], s_sh.at[dst_v.at[0]], sem_sc).wait()
        return carry

    lax.fori_loop(0, TPB, _drain_sc, 0)
    pltpu.sync_copy(ex_v, ex_hbm.at[pl.ds(base, TPB)])
    plsc.subcore_barrier()
    pltpu.sync_copy(s_sh.at[pl.ds(sid * nsub, nsub)],
                    spart_hbm.at[cid, pl.ds(sid * nsub, nsub)])


@functools.partial(
    pl.kernel,
    out_type=jax.ShapeDtypeStruct((NTILES, 16), jnp.float32),
    mesh=_mesh,
    compiler_params=_params,
    scratch_types=[
        pltpu.VMEM((4, N), jnp.float32),
        pltpu.VMEM((4, N), jnp.float32),
        pltpu.VMEM((NPAD,), jnp.float32),
        pltpu.VMEM((NPAD,), jnp.float32),
        [pltpu.VMEM((CH,), jnp.int32) for _ in range(2)],
        [pltpu.VMEM((CH,), jnp.int32) for _ in range(2)],
        [pltpu.VMEM((CH,), jnp.float32) for _ in range(2)],
        pltpu.VMEM((16,), jnp.float32),
        [pltpu.SemaphoreType.DMA for _ in range(2)],
    ],
)
def _sc_edge2(srcf_hbm, dstf_hbm, exf_hbm, spart_hbm, pt_hbm, qt_hbm, out_hbm,
              pt_v, qt_v, s_v, st_v, srcb, dstb, exb, orow, sems):
    cid = lax.axis_index("c")
    sid = lax.axis_index("s")
    wid = cid * 16 + sid
    ct = sid % 8
    eq = cid * 2 + sid // 8

    pltpu.sync_copy(pt_hbm.at[pl.ds(ct * 4, 4)], pt_v)
    pltpu.sync_copy(qt_hbm.at[pl.ds(ct * 4, 4)], qt_v)
    pltpu.sync_copy(spart_hbm.at[0], s_v)
    pltpu.sync_copy(spart_hbm.at[1], st_v)

    def _sum(i, carry):
        sl = pl.ds(i * 16, 16)
        s_v[sl] = 1.0 / (s_v[sl] + st_v[sl] + 1e-16)
        return carry

    lax.fori_loop(0, NPAD // 16, _sum, 0)

    ebase = eq * EQ_EDGES

    def _fire(ch, j):
        off = ebase + ch * CH
        pltpu.async_copy(srcf_hbm.at[pl.ds(off, CH)], srcb[j], sems[j])
        pltpu.async_copy(dstf_hbm.at[pl.ds(off, CH)], dstb[j], sems[j])
        pltpu.async_copy(exf_hbm.at[pl.ds(off, CH)], exb[j], sems[j])

    def _drain(j):
        pltpu.make_async_copy(srcf_hbm.at[pl.ds(0, CH)], srcb[j], sems[j]).wait()
        pltpu.make_async_copy(srcf_hbm.at[pl.ds(0, CH)], dstb[j], sems[j]).wait()
        pltpu.make_async_copy(exf_hbm.at[pl.ds(0, CH)], exb[j], sems[j]).wait()

    def _chunk(j, acc):
        def _group(k, acc):
            sl = pl.ds(k * 16, 16)
            srcv = srcb[j][sl]
            dstv = dstb[j][sl]
            alpha = exb[j][sl] * plsc.load_gather(s_v, [dstv])
            d0 = jnp.zeros((16,), jnp.float32)
            d1 = jnp.zeros((16,), jnp.float32)
            for cl in range(4):
                clsp = jnp.full((16,), cl, jnp.int32)
                pc = plsc.load_gather(pt_v, [clsp, srcv])
                qc = plsc.load_gather(qt_v, [clsp, dstv])
                if cl % 2 == 0:
                    d0 = d0 + pc * qc
                else:
                    d1 = d1 + pc * qc
            return acc + alpha * (d0 + d1)

        return lax.fori_loop(0, CH // 16, _group, acc)

    _fire(0, 0)

    def _loop(g, acc):
        for j in range(2):
            ch = 2 * g + j

            @pl.when(ch + 1 < NCH)
            def _():
                _fire(ch + 1, 1 - j)

            _drain(j)
            acc = _chunk(j, acc)
        return acc

    acc = lax.fori_loop(0, NCH // 2, _loop, jnp.zeros((16,), jnp.float32))
    orow[...] = acc * (-1.0 / N)
    pltpu.sync_copy(orow, out_hbm.at[wid])


def kernel(X, Mu, Var, edge_index, W, S, lin_l_w, lin_l_b, lin_r_w, lin_r_b, att):
    xl, xr = _tc_proj(X, lin_l_w, lin_l_b.reshape(1, H), lin_r_w,
                      lin_r_b.reshape(1, H))
    P, PT, QT, ll = _tc_dense(W, W.T, X, Mu, Var.reshape(1, G), S)
    xlr = jnp.concatenate([xl, xr], axis=1).astype(jnp.bfloat16)
    xlr = lax.bitcast_convert_type(xlr.reshape(N, H, 2), jnp.float32)
    pad = EPAD - E
    src = jnp.concatenate([edge_index[0], jnp.zeros((pad,), jnp.int32)])
    dst = jnp.concatenate([edge_index[1], jnp.zeros((pad,), jnp.int32)])
    att16 = jnp.pad(att, (0, 16 - H))
    ex, spart = _sc_edge1(src.reshape(ROWS, EPB), dst.reshape(ROWS, EPB),
                          xlr, att16)
    ce_part = _sc_edge2(src, dst, ex.reshape(EPAD), spart, PT, QT)
    return (ll[0, 0], jnp.sum(ce_part), P)


# speedup vs baseline: 3.0337x; 1.1918x over previous
import functools

import jax
import jax.numpy as jnp
from jax import lax
from jax.experimental import pallas as pl
from jax.experimental.pallas import tpu as pltpu
from jax.experimental.pallas import tpu_sc as plsc

N, C, G, E, H = 10000, 32, 128, 320000, 8
NPAD = 10240
EPB = 128
ROWS = 2560
EPAD = ROWS * EPB
NTILES = 32
TPB = ROWS // NTILES
CH = 1024
EO_EDGES = EPAD // 8
NCH = EO_EDGES // CH

_mesh = plsc.VectorSubcoreMesh(core_axis_name="c", subcore_axis_name="s")
_params = pltpu.CompilerParams(needs_layout_passes=False,
                               use_tc_tiling_on_sc=False)
_dn = (((1,), (1,)), ((), ()))


def _tc_proj_body(x_ref, wl_ref, bl_ref, wr_ref, br_ref, xl_ref, xr_ref):
    x = x_ref[...]
    xl_ref[...] = lax.dot_general(x, wl_ref[...], _dn,
                                  preferred_element_type=jnp.float32) + bl_ref[...]
    xr_ref[...] = lax.dot_general(x, wr_ref[...], _dn,
                                  preferred_element_type=jnp.float32) + br_ref[...]


_tc_proj = pl.pallas_call(
    _tc_proj_body,
    out_shape=[jax.ShapeDtypeStruct((N, H), jnp.float32),
               jax.ShapeDtypeStruct((N, H), jnp.float32)],
)


def _tc_dense_body(w_ref, wt_ref, x_ref, mu_ref, var_ref, s_ref,
                   p_ref, pt_ref, qt_ref, ll_ref):
    w = w_ref[...]
    m = jnp.max(w, axis=1, keepdims=True)
    e = jnp.exp(w - m)
    p = e / jnp.sum(e, axis=1, keepdims=True)
    p_ref[...] = p
    wt = wt_ref[...]
    mt = jnp.max(wt, axis=0, keepdims=True)
    et = jnp.exp(wt - mt)
    pt = et / jnp.sum(et, axis=0, keepdims=True)
    pt_ref[...] = pt
    qt_ref[...] = jnp.log(pt + 1e-8)
    iv = 1.0 / var_ref[...]
    x = x_ref[...]
    xv = x * iv
    A = lax.dot_general(xv, mu_ref[...], _dn,
                        preferred_element_type=jnp.float32)
    a = jnp.sum(x * xv, axis=1, keepdims=True)
    mu2 = mu_ref[...] * mu_ref[...]
    qrow = lax.dot_general(iv, mu2, _dn,
                           preferred_element_type=jnp.float32)
    s = s_ref[...]
    F = -0.5 * (a - 2.0 * s * A + (s * s) * qrow)
    ll_ref[...] = (jnp.sum(p * F) * (1.0 / N)).reshape(1, 1)


_tc_dense = pl.pallas_call(
    _tc_dense_body,
    out_shape=[jax.ShapeDtypeStruct((N, C), jnp.float32),
               jax.ShapeDtypeStruct((C, N), jnp.float32),
               jax.ShapeDtypeStruct((C, N), jnp.float32),
               jax.ShapeDtypeStruct((1, 1), jnp.float32)],
)


@functools.partial(
    pl.kernel,
    out_type=[jax.ShapeDtypeStruct((ROWS, EPB), jnp.float32),
              jax.ShapeDtypeStruct((2, NPAD), jnp.float32)],
    mesh=_mesh,
    compiler_params=_params,
    scratch_types=[
        pltpu.VMEM((N, H), jnp.float32),
        pltpu.VMEM((TPB, EPB), jnp.int32),
        pltpu.VMEM((TPB, EPB), jnp.int32),
        pltpu.VMEM((TPB, EPB), jnp.float32),
        pltpu.VMEM((16,), jnp.float32),
        pltpu.VMEM((NPAD // 16,), jnp.float32),
        pltpu.VMEM_SHARED((NPAD,), jnp.float32),
        pltpu.SemaphoreType.DMA,
        pltpu.SemaphoreType.DMA,
    ],
)
def _sc_edge1(src_hbm, dst_hbm, xlr_hbm, att_hbm, ex_hbm, spart_hbm,
              xlr_v, src_v, dst_v, ex_v, att_v, zbuf, s_sh, sem_sc, sem_p):
    cid = lax.axis_index("c")
    sid = lax.axis_index("s")
    wid = cid * 16 + sid
    base = wid * TPB
    nsub = NPAD // 16

    c1 = pltpu.async_copy(xlr_hbm, xlr_v, sem_p)
    c2 = pltpu.async_copy(src_hbm.at[pl.ds(base, TPB)], src_v, sem_p)
    c3 = pltpu.async_copy(dst_hbm.at[pl.ds(base, TPB)], dst_v, sem_p)
    c4 = pltpu.async_copy(att_hbm, att_v, sem_p)
    c1.wait()
    c2.wait()
    c3.wait()
    c4.wait()

    iota = lax.iota(jnp.int32, 16)
    zero16 = jnp.zeros((16,), jnp.float32)

    def _zero(i, carry):
        zbuf[pl.ds(i * 16, 16)] = zero16
        return carry

    lax.fori_loop(0, nsub // 16, _zero, 0)
    pltpu.sync_copy(zbuf, s_sh.at[pl.ds(sid * nsub, nsub)])
    plsc.subcore_barrier()

    att_full = att_v[...]
    bf = jnp.bfloat16
    fmt = plsc.PackFormat.INTERLEAVED

    def _row(b, carry):
        grow = base + b

        def _group(k, c2):
            sl = pl.ds(k * 16, 16)
            srcv = src_v[b, sl]
            dstv = dst_v[b, sl]
            acc_a = jnp.zeros((16,), jnp.float32)
            acc_b = jnp.zeros((16,), jnp.float32)
            for w in range(4):
                wl = plsc.load_gather(xlr_v, [srcv, jnp.full((16,), w, jnp.int32)])
                wr = plsc.load_gather(xlr_v, [dstv, jnp.full((16,), w + 4, jnp.int32)])
                la, lb = plsc.unpack(plsc.bitcast(wl, bf), format=fmt)
                ra, rb = plsc.unpack(plsc.bitcast(wr, bf), format=fmt)
                v0 = la + ra
                v1 = lb + rb
                a0 = att_full[2 * w]
                a1 = att_full[2 * w + 1]
                acc_a = acc_a + a0 * v0 + a1 * v1
                acc_b = acc_b + a0 * jnp.abs(v0) + a1 * jnp.abs(v1)
            exv = jnp.exp(0.6 * acc_a + 0.4 * acc_b)
            ids = iota + (grow * EPB + k * 16)
            exv = jnp.where(ids < E, exv, 0.0)
            ex_v[b, sl] = exv
            return c2

        lax.fori_loop(0, EPB // 16, _group, 0)
        pltpu.async_copy(ex_v.at[b], s_sh.at[dst_v.at[b]], sem_sc, add=True)
        return carry

    lax.fori_loop(0, TPB, _row, 0)

    def _drain_sc(b, carry):
        pltpu.make_async_copy(ex_v.at[0], s_sh.at[dst_v.at[0]], sem_sc).wait()
        return carry

    lax.fori_loop(0, TPB, _drain_sc, 0)
    pltpu.sync_copy(ex_v, ex_hbm.at[pl.ds(base, TPB)])
    plsc.subcore_barrier()
    pltpu.sync_copy(s_sh.at[pl.ds(sid * nsub, nsub)],
                    spart_hbm.at[cid, pl.ds(sid * nsub, nsub)])


@functools.partial(
    pl.kernel,
    out_type=jax.ShapeDtypeStruct((NTILES, 16), jnp.float32),
    mesh=_mesh,
    compiler_params=_params,
    scratch_types=[
        pltpu.VMEM((4, N), jnp.float32),
        pltpu.VMEM((4, N), jnp.float32),
        pltpu.VMEM((NPAD,), jnp.float32),
        pltpu.VMEM((NPAD,), jnp.float32),
        [pltpu.VMEM((CH,), jnp.int32) for _ in range(2)],
        [pltpu.VMEM((CH,), jnp.int32) for _ in range(2)],
        [pltpu.VMEM((CH,), jnp.float32) for _ in range(2)],
        pltpu.VMEM((16,), jnp.float32),
        [pltpu.SemaphoreType.DMA for _ in range(2)],
        pltpu.SemaphoreType.DMA,
    ],
)
def _sc_edge2(srcf_hbm, dstf_hbm, exf_hbm, spart_hbm, pt_hbm, qt_hbm, out_hbm,
              pt_v, qt_v, s_v, st_v, srcb, dstb, exb, orow, sems, sem_p):
    cid = lax.axis_index("c")
    sid = lax.axis_index("s")
    wid = cid * 16 + sid
    cq = sid % 4
    eo = wid // 4

    c1 = pltpu.async_copy(pt_hbm.at[pl.ds(cq * 4, 4)], pt_v, sem_p)
    c2 = pltpu.async_copy(qt_hbm.at[pl.ds(cq * 4, 4)], qt_v, sem_p)
    c3 = pltpu.async_copy(spart_hbm.at[0], s_v, sem_p)
    c4 = pltpu.async_copy(spart_hbm.at[1], st_v, sem_p)
    c1.wait()
    c2.wait()
    c3.wait()
    c4.wait()

    def _sum(i, carry):
        sl = pl.ds(i * 16, 16)
        s_v[sl] = 1.0 / (s_v[sl] + st_v[sl] + 1e-16)
        return carry

    lax.fori_loop(0, NPAD // 16, _sum, 0)

    ebase = eo * EO_EDGES

    def _fire(ch, j):
        off = ebase + ch * CH
        pltpu.async_copy(srcf_hbm.at[pl.ds(off, CH)], srcb[j], sems[j])
        pltpu.async_copy(dstf_hbm.at[pl.ds(off, CH)], dstb[j], sems[j])
        pltpu.async_copy(exf_hbm.at[pl.ds(off, CH)], exb[j], sems[j])

    def _drain(j):
        pltpu.make_async_copy(srcf_hbm.at[pl.ds(0, CH)], srcb[j], sems[j]).wait()
        pltpu.make_async_copy(srcf_hbm.at[pl.ds(0, CH)], dstb[j], sems[j]).wait()
        pltpu.make_async_copy(exf_hbm.at[pl.ds(0, CH)], exb[j], sems[j]).wait()

    bf = jnp.bfloat16
    fmt = plsc.PackFormat.INTERLEAVED

    def _chunk(j, acc):
        def _group2(k2, acc):
            for u in range(2):
                k = 2 * k2 + u
                sl = pl.ds(k * 16, 16)
                srcv = srcb[j][sl]
                dstv = dstb[j][sl]
                alpha = exb[j][sl] * plsc.load_gather(s_v, [dstv])
                d0 = jnp.zeros((16,), jnp.float32)
                d1 = jnp.zeros((16,), jnp.float32)
                for pw in range(4):
                    pwsp = jnp.full((16,), pw, jnp.int32)
                    pp = plsc.load_gather(pt_v, [pwsp, srcv])
                    qq = plsc.load_gather(qt_v, [pwsp, dstv])
                    pa, pb = plsc.unpack(plsc.bitcast(pp, bf), format=fmt)
                    qa, qb = plsc.unpack(plsc.bitcast(qq, bf), format=fmt)
                    d0 = d0 + pa * qa
                    d1 = d1 + pb * qb
                acc = acc + alpha * (d0 + d1)
            return acc

        return lax.fori_loop(0, CH // 32, _group2, acc)

    _fire(0, 0)

    def _loop(g, acc):
        for j in range(2):
            ch = 2 * g + j

            @pl.when(ch + 1 < NCH)
            def _():
                _fire(ch + 1, 1 - j)

            _drain(j)
            acc = _chunk(j, acc)
        return acc

    acc = lax.fori_loop(0, NCH // 2, _loop, jnp.zeros((16,), jnp.float32))
    orow[...] = acc * (-1.0 / N)
    pltpu.sync_copy(orow, out_hbm.at[wid])


def kernel(X, Mu, Var, edge_index, W, S, lin_l_w, lin_l_b, lin_r_w, lin_r_b, att):
    xl, xr = _tc_proj(X, lin_l_w, lin_l_b.reshape(1, H), lin_r_w,
                      lin_r_b.reshape(1, H))
    P, PT, QT, ll = _tc_dense(W, W.T, X, Mu, Var.reshape(1, G), S)
    xlr = jnp.concatenate([xl, xr], axis=1).astype(jnp.bfloat16)
    xlr = lax.bitcast_convert_type(xlr.reshape(N, H, 2), jnp.float32)
    pad = EPAD - E
    src = jnp.concatenate([edge_index[0], jnp.zeros((pad,), jnp.int32)])
    dst = jnp.concatenate([edge_index[1], jnp.zeros((pad,), jnp.int32)])
    att16 = jnp.pad(att, (0, 16 - H))
    ex, spart = _sc_edge1(src.reshape(ROWS, EPB), dst.reshape(ROWS, EPB),
                          xlr, att16)
    ptb = PT.astype(jnp.bfloat16)
    qtb = QT.astype(jnp.bfloat16)
    ptp = lax.bitcast_convert_type(jnp.stack([ptb[:16], ptb[16:]], axis=-1),
                                   jnp.float32)
    qtp = lax.bitcast_convert_type(jnp.stack([qtb[:16], qtb[16:]], axis=-1),
                                   jnp.float32)
    ce_part = _sc_edge2(src, dst, ex.reshape(EPAD), spart, ptp, qtp)
    return (ll[0, 0], jnp.sum(ce_part), P)
